# Initial kernel scaffold; baseline (speedup 1.0000x reference)
import jax
import jax.numpy as jnp
from jax.experimental import pallas as pl


def kernel(x, table, W1, b1, W2, b2):
    raise NotImplementedError("write your pallas kernel here")


# speedup vs baseline: 35.9941x; 35.9941x over previous
import functools

import jax
import jax.numpy as jnp
from jax import lax
from jax.experimental import pallas as pl
from jax.experimental.pallas import tpu as pltpu
from jax.experimental.pallas import tpu_sc as plsc

B = 16384
L = 500
D = 16
NC = 2
NS = 16
NW = NC * NS
BPW = B // NW
G = 16
NG = BPW // G


L2 = 2 * L
LPAD = 1024
PPW = BPW // 2
PPG = G // 2


def _sc_kernel(xf_hbm, table_hbm, w1_hbm, b1_hbm, w2_hbm, b2_hbm, out_hbm,
               xr0, xr1, rb0, rb1, pooled_v, zbuf, w1_v, b1_v, w2_v, b2_v,
               sem0, sem1):
    wid = lax.axis_index("s") * NC + lax.axis_index("c")
    base = wid * BPW

    pltpu.sync_copy(w1_hbm, w1_v)
    pltpu.sync_copy(b1_hbm, b1_v)
    pltpu.sync_copy(w2_hbm, w2_v)
    pltpu.sync_copy(b2_hbm, b2_v)

    zero16 = jnp.zeros((D,), jnp.float32)
    zi = jnp.zeros((D,), jnp.int32)
    xr0[pl.ds(L2, D)] = zi
    xr0[pl.ds(LPAD - D, D)] = zi
    xr1[pl.ds(L2, D)] = zi
    xr1[pl.ds(LPAD - D, D)] = zi

    xrows = (xr0, xr1)
    rbufs = (rb0, rb1)
    sems = (sem0, sem1)

    def stage_idx(pair, b):
        pltpu.sync_copy(xf_hbm.at[pl.ds(base * L + pair * L2, L2)],
                        xrows[b].at[pl.ds(0, L2)])

    def issue(b):
        return pltpu.async_copy(table_hbm.at[xrows[b]], rbufs[b], sems[b])

    def reduce_row(rb, off):
        def body(i, accs):
            a0, a1, a2, a3 = accs
            j = off + i * 4
            a0 = a0 + rb[j, :]
            a1 = a1 + rb[j + 1, :]
            a2 = a2 + rb[j + 2, :]
            a3 = a3 + rb[j + 3, :]
            return (a0, a1, a2, a3)

        a0, a1, a2, a3 = lax.fori_loop(0, L // 4, body,
                                       (zero16, zero16, zero16, zero16),
                                       unroll=4)
        return ((a0 + a1) + (a2 + a3)) * jnp.float32(1.0 / L)

    iota = lax.iota(jnp.int32, D)

    stage_idx(0, 0)
    issue(0)

    def per_group(g, _):
        for p in range(PPG):
            b, bn = p % 2, (p + 1) % 2
            pair = g * PPG + p

            @pl.when(pair + 1 < PPW)
            def _():
                stage_idx(pair + 1, bn)
                issue(bn)

            pltpu.make_async_copy(table_hbm.at[xrows[b]],
                                  rbufs[b], sems[b]).wait()
            pooled_v[2 * p, :] = reduce_row(rbufs[b], 0)
            pooled_v[2 * p + 1, :] = reduce_row(rbufs[b], L)

        def full(v):
            return jnp.full((D,), v, jnp.int32)

        pt = [plsc.load_gather(pooled_v, [iota, full(d)]) for d in range(D)]
        z = plsc.load_gather(b2_v, [full(0)])
        for j in range(D):
            h = plsc.load_gather(b1_v, [full(j)])
            for d in range(D):
                h = h + plsc.load_gather(w1_v, [full(d), full(j)]) * pt[d]
            h = jnp.maximum(h, jnp.float32(0.0))
            z = z + plsc.load_gather(w2_v, [full(j)]) * h
        zbuf[...] = jnp.float32(1.0) / (jnp.float32(1.0) + jnp.exp(-z))
        pltpu.sync_copy(zbuf, out_hbm.at[pl.ds(base + g * G, G)])
        return 0

    lax.fori_loop(0, NG, per_group, 0)


@jax.jit
def _run(x, table, w1, b1, w2, b2):
    mesh = plsc.VectorSubcoreMesh(core_axis_name="c", subcore_axis_name="s",
                                  num_cores=NC, num_subcores=NS)
    f = pl.kernel(
        _sc_kernel,
        out_type=jax.ShapeDtypeStruct((B,), jnp.float32),
        mesh=mesh,
        scratch_types=[
            pltpu.VMEM((LPAD,), jnp.int32),
            pltpu.VMEM((LPAD,), jnp.int32),
            pltpu.VMEM((LPAD, D), jnp.float32),
            pltpu.VMEM((LPAD, D), jnp.float32),
            pltpu.VMEM((G, D), jnp.float32),
            pltpu.VMEM((G,), jnp.float32),
            pltpu.VMEM((D, D), jnp.float32),
            pltpu.VMEM((D,), jnp.float32),
            pltpu.VMEM((D,), jnp.float32),
            pltpu.VMEM((D,), jnp.float32),
            pltpu.SemaphoreType.DMA,
            pltpu.SemaphoreType.DMA,
        ],
        compiler_params=pltpu.CompilerParams(needs_layout_passes=False,
                                             use_tc_tiling_on_sc=False),
    )
    return f(x, table, w1, b1, w2, b2)


def kernel(x, table, W1, b1, W2, b2):
    xf = x.reshape((B * L,))
    w2 = W2.reshape((D,))
    b2w = jnp.broadcast_to(b2, (D,))
    out = _run(xf, table, W1, b1, w2, b2w)
    return out.reshape((B, 1))
